# baseline (speedup 1.0000x reference)
import functools

import jax
import jax.numpy as jnp
from jax import lax
from jax.experimental import pallas as pl
from jax.experimental.pallas import tpu as pltpu
from jax.experimental.pallas import tpu_sc as plsc

BS = 16384
L = 32
P = 8
KSZ = 4
NUM_CH = 64
EMB_SIZE = 512
Z_DIM = 256
NUM_CODES = 128
COMMIT = 0.25

BLK = 1024
N_BLK = BS // BLK

NC = 2
NS = 16
LANES = 16
NW = NC * NS
B_PER_W = BS // NW
CH = 16
N_CHUNK = B_PER_W // CH
IDX_PER_CH = CH * P


def _main_body(fn_ref, valid_ref, wc_ref, cb_tiled_ref, codebook_ref, cbt_ref,
               eidx_ref, cmt_ref, perp_ref,
               hist_ref, acc_ref):
    i = pl.program_id(0)

    @pl.when(i == 0)
    def _init():
        hist_ref[...] = jnp.zeros_like(hist_ref)
        acc_ref[0] = 0.0
        acc_ref[1] = 0.0

    fn = fn_ref[...]
    valid = valid_ref[...]
    zbig = jnp.maximum(
        jnp.dot(fn, wc_ref[...], preferred_element_type=jnp.float32)
        + cb_tiled_ref[...], 0.0)

    codebook = codebook_ref[...]
    cbn2 = jnp.sum(codebook * codebook, axis=1)[None, :]
    iota = jax.lax.broadcasted_iota(jnp.int32, (BLK, NUM_CODES), 1)

    hist = jnp.zeros((1, NUM_CODES), jnp.float32)
    dsum = 0.0
    cols = []
    for p in range(P):
        z_p = zbig[:, p * NUM_CH:(p + 1) * NUM_CH]
        zn2 = jnp.sum(z_p * z_p, axis=1, keepdims=True)
        s_p = jnp.dot(z_p, cbt_ref[...], preferred_element_type=jnp.float32)
        dist = zn2 + cbn2 - 2.0 * s_p
        dmin = jnp.min(dist, axis=1, keepdims=True)
        idx = jnp.min(jnp.where(dist == dmin, iota, NUM_CODES), axis=1,
                      keepdims=True)
        oh = (iota == idx).astype(jnp.float32)
        hist = hist + jnp.sum(oh, axis=0, keepdims=True)
        dsum = dsum + jnp.sum(dmin * valid)
        cols.append(idx + p * NUM_CODES)
    eidx_ref[...] = jnp.concatenate(cols, axis=1)

    hist_ref[...] += hist
    acc_ref[0] += dsum
    acc_ref[1] += jnp.sum(valid)

    @pl.when(i == N_BLK - 1)
    def _fini():
        denom = jnp.maximum(acc_ref[1] * (P * NUM_CH), 1.0)
        cmt_ref[...] = jnp.full((1, 1), COMMIT * acc_ref[0] / denom,
                                jnp.float32)
        avgp = hist_ref[...] / float(BS * P)
        perp_ref[...] = jnp.full(
            (1, 1), jnp.exp(-jnp.sum(avgp * jnp.log(avgp + 1e-10))),
            jnp.float32)


def _mall_body(cb_ref, fc_wpt_ref, mu_wt_ref, fc_b_ref, mu_b_ref, mall_ref):
    t = jnp.dot(cb_ref[...], fc_wpt_ref[0], preferred_element_type=jnp.float32)
    m = jnp.dot(t, mu_wt_ref[...], preferred_element_type=jnp.float32)
    bias = (jnp.dot(fc_b_ref[...], mu_wt_ref[...],
                    preferred_element_type=jnp.float32)
            + mu_b_ref[...]) * (1.0 / P)
    mall_ref[...] = m + bias


def _sc_gather_body(eidx_hbm, mall_hbm, out_hbm, idx_all_v, rows_v, out_v,
                    sem0, sem1):
    wid = lax.axis_index("s") * NC + lax.axis_index("c")
    base = wid * B_PER_W

    pltpu.sync_copy(eidx_hbm.at[pl.ds(wid * N_CHUNK, N_CHUNK)], idx_all_v)
    pltpu.async_copy(mall_hbm.at[idx_all_v.at[0]], rows_v.at[0], sem0)
    pltpu.async_copy(mall_hbm.at[idx_all_v.at[1]], rows_v.at[1], sem1)

    def pair(k0, carry):
        for b in range(2):
            k = k0 * 2 + b
            sem = sem0 if b == 0 else sem1
            pltpu.make_async_copy(mall_hbm.at[idx_all_v.at[k]],
                                  rows_v.at[b], sem).wait()

            @plsc.parallel_loop(0, CH, 1)
            def _sample(s):
                for g in range(Z_DIM // LANES):
                    sl = pl.ds(g * LANES, LANES)
                    acc = rows_v[b, s * P, sl]
                    for p in range(1, P):
                        acc = acc + rows_v[b, s * P + p, sl]
                    out_v[b, s, sl] = acc

            pltpu.sync_copy(out_v.at[b], out_hbm.at[pl.ds(base + k * CH, CH)])

            @pl.when(k + 2 < N_CHUNK)
            def _fire_next():
                pltpu.async_copy(mall_hbm.at[idx_all_v.at[k + 2]],
                                 rows_v.at[b], sem)
        return carry

    lax.fori_loop(0, N_CHUNK // 2, pair, 0)


@jax.jit
def kernel(fn, track_pad_mask, conv_w, conv_b, fc_w, fc_b, mu_w, mu_b, codebook):
    valid = 1.0 - track_pad_mask.astype(jnp.float32)
    w_kc = conv_w[:, 0, :].T
    wc = jnp.kron(jnp.eye(P, dtype=jnp.float32), w_kc)
    cb_tiled = jnp.tile(conv_b, P)[None, :]
    fc_wpt = fc_w.reshape(EMB_SIZE, NUM_CH, P).transpose(2, 1, 0)
    mu_wt = mu_w.T

    eidx, cmt, perp = pl.pallas_call(
        _main_body,
        grid=(N_BLK,),
        in_specs=[
            pl.BlockSpec((BLK, L), lambda i: (i, 0)),
            pl.BlockSpec((BLK, 1), lambda i: (i, 0)),
            pl.BlockSpec((L, EMB_SIZE), lambda i: (0, 0)),
            pl.BlockSpec((1, EMB_SIZE), lambda i: (0, 0)),
            pl.BlockSpec((NUM_CODES, NUM_CH), lambda i: (0, 0)),
            pl.BlockSpec((NUM_CH, NUM_CODES), lambda i: (0, 0)),
        ],
        out_specs=[
            pl.BlockSpec((BLK, P), lambda i: (i, 0)),
            pl.BlockSpec((1, 1), lambda i: (0, 0)),
            pl.BlockSpec((1, 1), lambda i: (0, 0)),
        ],
        out_shape=[
            jax.ShapeDtypeStruct((BS, P), jnp.int32),
            jax.ShapeDtypeStruct((1, 1), jnp.float32),
            jax.ShapeDtypeStruct((1, 1), jnp.float32),
        ],
        scratch_shapes=[
            pltpu.VMEM((1, NUM_CODES), jnp.float32),
            pltpu.SMEM((2,), jnp.float32),
        ],
        compiler_params=pltpu.CompilerParams(
            dimension_semantics=("arbitrary",)),
    )(fn, valid, wc, cb_tiled, codebook, codebook.T)

    mall = pl.pallas_call(
        _mall_body,
        grid=(P,),
        in_specs=[
            pl.BlockSpec((NUM_CODES, NUM_CH), lambda p: (0, 0)),
            pl.BlockSpec((1, NUM_CH, EMB_SIZE), lambda p: (p, 0, 0)),
            pl.BlockSpec((EMB_SIZE, Z_DIM), lambda p: (0, 0)),
            pl.BlockSpec((1, EMB_SIZE), lambda p: (0, 0)),
            pl.BlockSpec((1, Z_DIM), lambda p: (0, 0)),
        ],
        out_specs=pl.BlockSpec((NUM_CODES, Z_DIM), lambda p: (p, 0)),
        out_shape=jax.ShapeDtypeStruct((P * NUM_CODES, Z_DIM), jnp.float32),
        compiler_params=pltpu.CompilerParams(
            dimension_semantics=("arbitrary",)),
    )(codebook, fc_wpt, mu_wt, fc_b[None, :], mu_b[None, :])

    sc_gather = pl.kernel(
        _sc_gather_body,
        out_type=jax.ShapeDtypeStruct((BS, Z_DIM), jnp.float32),
        mesh=plsc.VectorSubcoreMesh(core_axis_name="c", subcore_axis_name="s"),
        scratch_types=[
            pltpu.VMEM((N_CHUNK, IDX_PER_CH), jnp.int32),
            pltpu.VMEM((2, IDX_PER_CH, Z_DIM), jnp.float32),
            pltpu.VMEM((2, CH, Z_DIM), jnp.float32),
            pltpu.SemaphoreType.DMA,
            pltpu.SemaphoreType.DMA,
        ],
    )
    mu = sc_gather(eidx.reshape(NW * N_CHUNK, IDX_PER_CH), mall)

    return mu, cmt.reshape(()), perp.reshape(())


# speedup vs baseline: 1.1804x; 1.1804x over previous
import functools

import jax
import jax.numpy as jnp
from jax import lax
from jax.experimental import pallas as pl
from jax.experimental.pallas import tpu as pltpu
from jax.experimental.pallas import tpu_sc as plsc

BS = 16384
L = 32
P = 8
KSZ = 4
NUM_CH = 64
EMB_SIZE = 512
Z_DIM = 256
NUM_CODES = 128
COMMIT = 0.25

BLK = 1024
N_BLK = BS // BLK

NC = 2
NS = 16
LANES = 16
NW = NC * NS
B_PER_W = BS // NW
CH = 16
N_CHUNK = B_PER_W // CH
IDX_PER_CH = CH * P


def _main_body(fn_ref, valid_ref, wc_ref, cb_tiled_ref, codebook_ref, cbt_ref,
               eidx_ref, cmt_ref, perp_ref,
               hist_ref, acc_ref):
    i = pl.program_id(0)

    @pl.when(i == 0)
    def _init():
        hist_ref[...] = jnp.zeros_like(hist_ref)
        acc_ref[0] = 0.0
        acc_ref[1] = 0.0

    fn = fn_ref[...]
    valid = valid_ref[...]
    zbig = jnp.maximum(
        jnp.dot(fn, wc_ref[...], preferred_element_type=jnp.float32)
        + cb_tiled_ref[...], 0.0)

    codebook = codebook_ref[...]
    cbn2 = jnp.sum(codebook * codebook, axis=1)[None, :]
    iota = jax.lax.broadcasted_iota(jnp.int32, (BLK, NUM_CODES), 1)

    hist = jnp.zeros((1, NUM_CODES), jnp.float32)
    dsum = 0.0
    cols = []
    for p in range(P):
        z_p = zbig[:, p * NUM_CH:(p + 1) * NUM_CH]
        zn2 = jnp.sum(z_p * z_p, axis=1, keepdims=True)
        s_p = jnp.dot(z_p, cbt_ref[...], preferred_element_type=jnp.float32)
        dist = zn2 + cbn2 - 2.0 * s_p
        dmin = jnp.min(dist, axis=1, keepdims=True)
        idx = jnp.min(jnp.where(dist == dmin, iota, NUM_CODES), axis=1,
                      keepdims=True)
        oh = (iota == idx).astype(jnp.float32)
        hist = hist + jnp.sum(oh, axis=0, keepdims=True)
        dsum = dsum + jnp.sum(dmin * valid)
        cols.append(idx + p * NUM_CODES)
    eidx_ref[...] = jnp.concatenate(cols, axis=1)

    hist_ref[...] += hist
    acc_ref[0] += dsum
    acc_ref[1] += jnp.sum(valid)

    @pl.when(i == N_BLK - 1)
    def _fini():
        denom = jnp.maximum(acc_ref[1] * (P * NUM_CH), 1.0)
        cmt_ref[...] = jnp.full((1, 1), COMMIT * acc_ref[0] / denom,
                                jnp.float32)
        avgp = hist_ref[...] / float(BS * P)
        perp_ref[...] = jnp.full(
            (1, 1), jnp.exp(-jnp.sum(avgp * jnp.log(avgp + 1e-10))),
            jnp.float32)


def _mall_body(cb_ref, fc_wpt_ref, mu_wt_ref, fc_b_ref, mu_b_ref, mall_ref):
    t = jnp.dot(cb_ref[...], fc_wpt_ref[0], preferred_element_type=jnp.float32)
    m = jnp.dot(t, mu_wt_ref[...], preferred_element_type=jnp.float32)
    bias = (jnp.dot(fc_b_ref[...], mu_wt_ref[...],
                    preferred_element_type=jnp.float32)
            + mu_b_ref[...]) * (1.0 / P)
    mall_ref[...] = m + bias


HALF = BS // NC
CCH = 2048
N_CCH = HALF // CCH
COLS = Z_DIM // NS


def _sc_gather_body(eidx_hbm, mall_hbm, out_hbm, table_v, eidx_v, out_v):
    core = lax.axis_index("c")
    tile = lax.axis_index("s")
    sample_base = core * HALF

    pltpu.sync_copy(
        mall_hbm.at[pl.ds(tile * (P * NUM_CODES * COLS), P * NUM_CODES * COLS)],
        table_v)

    def chunk(k, carry):
        s0 = sample_base + k * CCH
        pltpu.sync_copy(eidx_hbm.at[:, pl.ds(s0, CCH)], eidx_v)

        @plsc.parallel_loop(0, CCH // LANES, 1)
        def _group(g):
            ev = [eidx_v[p, pl.ds(g * LANES, LANES)] * COLS for p in range(P)]
            for c in range(COLS):
                acc = plsc.load_gather(table_v, [ev[0] + c])
                for p in range(1, P):
                    acc = acc + plsc.load_gather(table_v, [ev[p] + c])
                out_v[c, pl.ds(g * LANES, LANES)] = acc

        pltpu.sync_copy(out_v, out_hbm.at[tile, :, pl.ds(s0, CCH)])
        return carry

    lax.fori_loop(0, N_CCH, chunk, 0)


@jax.jit
def kernel(fn, track_pad_mask, conv_w, conv_b, fc_w, fc_b, mu_w, mu_b, codebook):
    valid = 1.0 - track_pad_mask.astype(jnp.float32)
    w_kc = conv_w[:, 0, :].T
    wc = jnp.kron(jnp.eye(P, dtype=jnp.float32), w_kc)
    cb_tiled = jnp.tile(conv_b, P)[None, :]
    fc_wpt = fc_w.reshape(EMB_SIZE, NUM_CH, P).transpose(2, 1, 0)
    mu_wt = mu_w.T

    eidx, cmt, perp = pl.pallas_call(
        _main_body,
        grid=(N_BLK,),
        in_specs=[
            pl.BlockSpec((BLK, L), lambda i: (i, 0)),
            pl.BlockSpec((BLK, 1), lambda i: (i, 0)),
            pl.BlockSpec((L, EMB_SIZE), lambda i: (0, 0)),
            pl.BlockSpec((1, EMB_SIZE), lambda i: (0, 0)),
            pl.BlockSpec((NUM_CODES, NUM_CH), lambda i: (0, 0)),
            pl.BlockSpec((NUM_CH, NUM_CODES), lambda i: (0, 0)),
        ],
        out_specs=[
            pl.BlockSpec((BLK, P), lambda i: (i, 0)),
            pl.BlockSpec((1, 1), lambda i: (0, 0)),
            pl.BlockSpec((1, 1), lambda i: (0, 0)),
        ],
        out_shape=[
            jax.ShapeDtypeStruct((BS, P), jnp.int32),
            jax.ShapeDtypeStruct((1, 1), jnp.float32),
            jax.ShapeDtypeStruct((1, 1), jnp.float32),
        ],
        scratch_shapes=[
            pltpu.VMEM((1, NUM_CODES), jnp.float32),
            pltpu.SMEM((2,), jnp.float32),
        ],
        compiler_params=pltpu.CompilerParams(
            dimension_semantics=("arbitrary",)),
    )(fn, valid, wc, cb_tiled, codebook, codebook.T)

    mall = pl.pallas_call(
        _mall_body,
        grid=(P,),
        in_specs=[
            pl.BlockSpec((NUM_CODES, NUM_CH), lambda p: (0, 0)),
            pl.BlockSpec((1, NUM_CH, EMB_SIZE), lambda p: (p, 0, 0)),
            pl.BlockSpec((EMB_SIZE, Z_DIM), lambda p: (0, 0)),
            pl.BlockSpec((1, EMB_SIZE), lambda p: (0, 0)),
            pl.BlockSpec((1, Z_DIM), lambda p: (0, 0)),
        ],
        out_specs=pl.BlockSpec((NUM_CODES, Z_DIM), lambda p: (p, 0)),
        out_shape=jax.ShapeDtypeStruct((P * NUM_CODES, Z_DIM), jnp.float32),
        compiler_params=pltpu.CompilerParams(
            dimension_semantics=("arbitrary",)),
    )(codebook, fc_wpt, mu_wt, fc_b[None, :], mu_b[None, :])

    mall_cs = mall.reshape(P * NUM_CODES, NS, COLS).transpose(1, 0, 2).reshape(-1)

    sc_gather = pl.kernel(
        _sc_gather_body,
        out_type=jax.ShapeDtypeStruct((NS, COLS, BS), jnp.float32),
        mesh=plsc.VectorSubcoreMesh(core_axis_name="c", subcore_axis_name="s"),
        scratch_types=[
            pltpu.VMEM((P * NUM_CODES * COLS,), jnp.float32),
            pltpu.VMEM((P, CCH), jnp.int32),
            pltpu.VMEM((COLS, CCH), jnp.float32),
        ],
        compiler_params=pltpu.CompilerParams(needs_layout_passes=False),
    )
    out3 = sc_gather(eidx.T, mall_cs)
    mu = out3.transpose(2, 0, 1).reshape(BS, Z_DIM)

    return mu, cmt.reshape(()), perp.reshape(())


# speedup vs baseline: 2.7908x; 2.3644x over previous
import functools

import jax
import jax.numpy as jnp
from jax import lax
from jax.experimental import pallas as pl
from jax.experimental.pallas import tpu as pltpu
from jax.experimental.pallas import tpu_sc as plsc

BS = 16384
L = 32
P = 8
KSZ = 4
NUM_CH = 64
EMB_SIZE = 512
Z_DIM = 256
NUM_CODES = 128
COMMIT = 0.25

BLK = 1024
N_BLK = BS // BLK

NC = 2
NS = 16
LANES = 16
NW = NC * NS
B_PER_W = BS // NW
CH = 16
N_CHUNK = B_PER_W // CH
IDX_PER_CH = CH * P


def _main_body(fn_ref, valid_ref, wc_ref, cb_tiled_ref, codebook_ref, cbt_ref,
               eidx_ref, cmt_ref, perp_ref,
               hist_ref, acc_ref):
    i = pl.program_id(0)

    @pl.when(i == 0)
    def _init():
        hist_ref[...] = jnp.zeros_like(hist_ref)
        acc_ref[0] = 0.0
        acc_ref[1] = 0.0

    fn = fn_ref[...]
    valid = valid_ref[...]
    zbig = jnp.maximum(
        jnp.dot(fn, wc_ref[...], preferred_element_type=jnp.float32)
        + cb_tiled_ref[...], 0.0)

    codebook = codebook_ref[...]
    cbn2 = jnp.sum(codebook * codebook, axis=1)[None, :]
    iota = jax.lax.broadcasted_iota(jnp.int32, (BLK, NUM_CODES), 1)

    hist = jnp.zeros((1, NUM_CODES), jnp.float32)
    dsum = 0.0
    cols = []
    for p in range(P):
        z_p = zbig[:, p * NUM_CH:(p + 1) * NUM_CH]
        zn2 = jnp.sum(z_p * z_p, axis=1, keepdims=True)
        s_p = jnp.dot(z_p, cbt_ref[...], preferred_element_type=jnp.float32)
        dist = zn2 + cbn2 - 2.0 * s_p
        dmin = jnp.min(dist, axis=1, keepdims=True)
        idx = jnp.min(jnp.where(dist == dmin, iota, NUM_CODES), axis=1,
                      keepdims=True)
        oh = (iota == idx).astype(jnp.float32)
        hist = hist + jnp.sum(oh, axis=0, keepdims=True)
        dsum = dsum + jnp.sum(dmin * valid)
        cols.append(idx + p * NUM_CODES)
    eidx_ref[...] = jnp.concatenate(cols, axis=1)

    hist_ref[...] += hist
    acc_ref[0] += dsum
    acc_ref[1] += jnp.sum(valid)

    @pl.when(i == N_BLK - 1)
    def _fini():
        denom = jnp.maximum(acc_ref[1] * (P * NUM_CH), 1.0)
        cmt_ref[...] = jnp.full((1, 1), COMMIT * acc_ref[0] / denom,
                                jnp.float32)
        avgp = hist_ref[...] / float(BS * P)
        perp_ref[...] = jnp.full(
            (1, 1), jnp.exp(-jnp.sum(avgp * jnp.log(avgp + 1e-10))),
            jnp.float32)


def _mall_body(cb_ref, fc_wpt_ref, mu_wt_ref, fc_b_ref, mu_b_ref, mall_ref):
    t = jnp.dot(cb_ref[...], fc_wpt_ref[0], preferred_element_type=jnp.float32)
    m = jnp.dot(t, mu_wt_ref[...], preferred_element_type=jnp.float32)
    bias = (jnp.dot(fc_b_ref[...], mu_wt_ref[...],
                    preferred_element_type=jnp.float32)
            + mu_b_ref[...]) * (1.0 / P)
    mall_ref[...] = m + bias


HALF = BS // NC
CCH = 2048
N_CCH = HALF // CCH
COLS = Z_DIM // NS


def _sc_gather_body(eidx_hbm, mall_hbm, out_hbm, table_v, eidx_v, out_v):
    core = lax.axis_index("c")
    tile = lax.axis_index("s")
    sample_base = core * HALF

    pltpu.sync_copy(
        mall_hbm.at[pl.ds(tile * (P * NUM_CODES * COLS), P * NUM_CODES * COLS)],
        table_v)

    def chunk(k, carry):
        s0 = sample_base + k * CCH
        pltpu.sync_copy(eidx_hbm.at[:, pl.ds(s0, CCH)], eidx_v)

        @plsc.parallel_loop(0, CCH // LANES, 1)
        def _group(g):
            ev = [eidx_v[p, pl.ds(g * LANES, LANES)] for p in range(P)]
            for c in range(COLS):
                cbase = c * (P * NUM_CODES)
                acc = plsc.load_gather(table_v, [ev[0] + cbase])
                for p in range(1, P):
                    acc = acc + plsc.load_gather(table_v, [ev[p] + cbase])
                out_v[c, pl.ds(g * LANES, LANES)] = acc

        pltpu.sync_copy(out_v, out_hbm.at[tile, :, pl.ds(s0, CCH)])
        return carry

    lax.fori_loop(0, N_CCH, chunk, 0)


@jax.jit
def kernel(fn, track_pad_mask, conv_w, conv_b, fc_w, fc_b, mu_w, mu_b, codebook):
    valid = 1.0 - track_pad_mask.astype(jnp.float32)
    w_kc = conv_w[:, 0, :].T
    wc = jnp.kron(jnp.eye(P, dtype=jnp.float32), w_kc)
    cb_tiled = jnp.tile(conv_b, P)[None, :]
    fc_wpt = fc_w.reshape(EMB_SIZE, NUM_CH, P).transpose(2, 1, 0)
    mu_wt = mu_w.T

    eidx, cmt, perp = pl.pallas_call(
        _main_body,
        grid=(N_BLK,),
        in_specs=[
            pl.BlockSpec((BLK, L), lambda i: (i, 0)),
            pl.BlockSpec((BLK, 1), lambda i: (i, 0)),
            pl.BlockSpec((L, EMB_SIZE), lambda i: (0, 0)),
            pl.BlockSpec((1, EMB_SIZE), lambda i: (0, 0)),
            pl.BlockSpec((NUM_CODES, NUM_CH), lambda i: (0, 0)),
            pl.BlockSpec((NUM_CH, NUM_CODES), lambda i: (0, 0)),
        ],
        out_specs=[
            pl.BlockSpec((BLK, P), lambda i: (i, 0)),
            pl.BlockSpec((1, 1), lambda i: (0, 0)),
            pl.BlockSpec((1, 1), lambda i: (0, 0)),
        ],
        out_shape=[
            jax.ShapeDtypeStruct((BS, P), jnp.int32),
            jax.ShapeDtypeStruct((1, 1), jnp.float32),
            jax.ShapeDtypeStruct((1, 1), jnp.float32),
        ],
        scratch_shapes=[
            pltpu.VMEM((1, NUM_CODES), jnp.float32),
            pltpu.SMEM((2,), jnp.float32),
        ],
        compiler_params=pltpu.CompilerParams(
            dimension_semantics=("arbitrary",)),
    )(fn, valid, wc, cb_tiled, codebook, codebook.T)

    mall = pl.pallas_call(
        _mall_body,
        grid=(P,),
        in_specs=[
            pl.BlockSpec((NUM_CODES, NUM_CH), lambda p: (0, 0)),
            pl.BlockSpec((1, NUM_CH, EMB_SIZE), lambda p: (p, 0, 0)),
            pl.BlockSpec((EMB_SIZE, Z_DIM), lambda p: (0, 0)),
            pl.BlockSpec((1, EMB_SIZE), lambda p: (0, 0)),
            pl.BlockSpec((1, Z_DIM), lambda p: (0, 0)),
        ],
        out_specs=pl.BlockSpec((NUM_CODES, Z_DIM), lambda p: (p, 0)),
        out_shape=jax.ShapeDtypeStruct((P * NUM_CODES, Z_DIM), jnp.float32),
        compiler_params=pltpu.CompilerParams(
            dimension_semantics=("arbitrary",)),
    )(codebook, fc_wpt, mu_wt, fc_b[None, :], mu_b[None, :])

    mall_cs = mall.reshape(P * NUM_CODES, NS, COLS).transpose(1, 2, 0).reshape(-1)

    sc_gather = pl.kernel(
        _sc_gather_body,
        out_type=jax.ShapeDtypeStruct((NS, COLS, BS), jnp.float32),
        mesh=plsc.VectorSubcoreMesh(core_axis_name="c", subcore_axis_name="s"),
        scratch_types=[
            pltpu.VMEM((P * NUM_CODES * COLS,), jnp.float32),
            pltpu.VMEM((P, CCH), jnp.int32),
            pltpu.VMEM((COLS, CCH), jnp.float32),
        ],
        compiler_params=pltpu.CompilerParams(needs_layout_passes=False),
    )
    out3 = sc_gather(eidx.T, mall_cs)
    mu = out3.transpose(2, 0, 1).reshape(BS, Z_DIM)

    return mu, cmt.reshape(()), perp.reshape(())


# speedup vs baseline: 3.5648x; 1.2773x over previous
import functools

import jax
import jax.numpy as jnp
from jax import lax
from jax.experimental import pallas as pl
from jax.experimental.pallas import tpu as pltpu
from jax.experimental.pallas import tpu_sc as plsc

BS = 16384
L = 32
P = 8
KSZ = 4
NUM_CH = 64
EMB_SIZE = 512
Z_DIM = 256
NUM_CODES = 128
COMMIT = 0.25

BLK = 1024
N_BLK = BS // BLK

NC = 2
NS = 16
LANES = 16
NW = NC * NS
B_PER_W = BS // NW
CH = 16
N_CHUNK = B_PER_W // CH
IDX_PER_CH = CH * P


def _main_body(fn_ref, valid_ref, wc_ref, cb_tiled_ref, codebook_ref, cbt_ref,
               bones_ref, ones_row_ref,
               eidx_ref, cmt_ref, perp_ref,
               hist_ref, acc_ref):
    i = pl.program_id(0)

    @pl.when(i == 0)
    def _init():
        hist_ref[...] = jnp.zeros_like(hist_ref)
        acc_ref[0] = 0.0
        acc_ref[1] = 0.0

    fn = fn_ref[...]
    valid = valid_ref[...]
    zbig = jnp.maximum(
        jnp.dot(fn, wc_ref[...], preferred_element_type=jnp.float32)
        + cb_tiled_ref[...], 0.0)

    codebook = codebook_ref[...]
    cbn2 = jnp.sum(codebook * codebook, axis=1)[None, :]
    cbtm2 = cbt_ref[...] * -2.0
    iotaf = jax.lax.broadcasted_iota(
        jnp.int32, (BLK, NUM_CODES), 1).astype(jnp.float32)

    zn2_all = jnp.dot(zbig * zbig, bones_ref[...],
                      preferred_element_type=jnp.float32)
    zn2_row = jnp.sum(zn2_all, axis=1, keepdims=True)

    oh_sum = jnp.zeros((BLK, NUM_CODES), jnp.float32)
    dvec = jnp.zeros((BLK, 1), jnp.float32)
    cols = []
    for p in range(P):
        z_p = zbig[:, p * NUM_CH:(p + 1) * NUM_CH]
        dist = jnp.dot(z_p, cbtm2, preferred_element_type=jnp.float32) + cbn2
        dmin = jnp.min(dist, axis=1, keepdims=True)
        eqm = dist == dmin
        idxf = jnp.min(jnp.where(eqm, iotaf, 1e9), axis=1, keepdims=True)
        oh_sum = oh_sum + eqm.astype(jnp.float32)
        dvec = dvec + dmin
        cols.append(idxf.astype(jnp.int32) + p * NUM_CODES)
    eidx_ref[...] = jnp.concatenate(cols, axis=1)

    hist_ref[...] += jnp.dot(ones_row_ref[...], oh_sum,
                             preferred_element_type=jnp.float32)
    acc_ref[0] += jnp.sum((dvec + zn2_row) * valid)
    acc_ref[1] += jnp.sum(valid)

    @pl.when(i == N_BLK - 1)
    def _fini():
        denom = jnp.maximum(acc_ref[1] * (P * NUM_CH), 1.0)
        cmt_ref[...] = jnp.full((1, 1), COMMIT * acc_ref[0] / denom,
                                jnp.float32)
        avgp = hist_ref[...] / float(BS * P)
        perp_ref[...] = jnp.full(
            (1, 1), jnp.exp(-jnp.sum(avgp * jnp.log(avgp + 1e-10))),
            jnp.float32)


def _mall_body(cb_ref, fc_wpt_ref, mu_wt_ref, fc_b_ref, mu_b_ref, mall_ref):
    t = jnp.dot(cb_ref[...], fc_wpt_ref[0], preferred_element_type=jnp.float32)
    m = jnp.dot(t, mu_wt_ref[...], preferred_element_type=jnp.float32)
    bias = (jnp.dot(fc_b_ref[...], mu_wt_ref[...],
                    preferred_element_type=jnp.float32)
            + mu_b_ref[...]) * (1.0 / P)
    mall_ref[...] = m + bias


HALF = BS // NC
CCH = 2048
N_CCH = HALF // CCH
COLS = Z_DIM // NS


def _sc_gather_body(eidx_hbm, mall_hbm, out_hbm, table_v, eidx_v, out_v):
    core = lax.axis_index("c")
    tile = lax.axis_index("s")
    sample_base = core * HALF

    pltpu.sync_copy(
        mall_hbm.at[pl.ds(tile * (P * NUM_CODES * COLS), P * NUM_CODES * COLS)],
        table_v)

    def chunk(k, carry):
        s0 = sample_base + k * CCH
        pltpu.sync_copy(eidx_hbm.at[:, pl.ds(s0, CCH)], eidx_v)

        @plsc.parallel_loop(0, CCH // LANES, 1)
        def _group(g):
            ev = [eidx_v[p, pl.ds(g * LANES, LANES)] for p in range(P)]
            for c in range(COLS):
                cbase = c * (P * NUM_CODES)
                acc = plsc.load_gather(table_v, [ev[0] + cbase])
                for p in range(1, P):
                    acc = acc + plsc.load_gather(table_v, [ev[p] + cbase])
                out_v[c, pl.ds(g * LANES, LANES)] = acc

        pltpu.sync_copy(out_v, out_hbm.at[tile, :, pl.ds(s0, CCH)])
        return carry

    lax.fori_loop(0, N_CCH, chunk, 0)


@jax.jit
def kernel(fn, track_pad_mask, conv_w, conv_b, fc_w, fc_b, mu_w, mu_b, codebook):
    valid = 1.0 - track_pad_mask.astype(jnp.float32)
    w_kc = conv_w[:, 0, :].T
    wc = jnp.kron(jnp.eye(P, dtype=jnp.float32), w_kc)
    cb_tiled = jnp.tile(conv_b, P)[None, :]
    fc_wpt = fc_w.reshape(EMB_SIZE, NUM_CH, P).transpose(2, 1, 0)
    mu_wt = mu_w.T

    eidx, cmt, perp = pl.pallas_call(
        _main_body,
        grid=(N_BLK,),
        in_specs=[
            pl.BlockSpec((BLK, L), lambda i: (i, 0)),
            pl.BlockSpec((BLK, 1), lambda i: (i, 0)),
            pl.BlockSpec((L, EMB_SIZE), lambda i: (0, 0)),
            pl.BlockSpec((1, EMB_SIZE), lambda i: (0, 0)),
            pl.BlockSpec((NUM_CODES, NUM_CH), lambda i: (0, 0)),
            pl.BlockSpec((NUM_CH, NUM_CODES), lambda i: (0, 0)),
            pl.BlockSpec((EMB_SIZE, P), lambda i: (0, 0)),
            pl.BlockSpec((1, BLK), lambda i: (0, 0)),
        ],
        out_specs=[
            pl.BlockSpec((BLK, P), lambda i: (i, 0)),
            pl.BlockSpec((1, 1), lambda i: (0, 0)),
            pl.BlockSpec((1, 1), lambda i: (0, 0)),
        ],
        out_shape=[
            jax.ShapeDtypeStruct((BS, P), jnp.int32),
            jax.ShapeDtypeStruct((1, 1), jnp.float32),
            jax.ShapeDtypeStruct((1, 1), jnp.float32),
        ],
        scratch_shapes=[
            pltpu.VMEM((1, NUM_CODES), jnp.float32),
            pltpu.SMEM((2,), jnp.float32),
        ],
        compiler_params=pltpu.CompilerParams(
            dimension_semantics=("arbitrary",)),
    )(fn, valid, wc, cb_tiled, codebook, codebook.T,
      jnp.kron(jnp.eye(P, dtype=jnp.float32), jnp.ones((NUM_CH, 1), jnp.float32)),
      jnp.ones((1, BLK), jnp.float32))

    mall = pl.pallas_call(
        _mall_body,
        grid=(P,),
        in_specs=[
            pl.BlockSpec((NUM_CODES, NUM_CH), lambda p: (0, 0)),
            pl.BlockSpec((1, NUM_CH, EMB_SIZE), lambda p: (p, 0, 0)),
            pl.BlockSpec((EMB_SIZE, Z_DIM), lambda p: (0, 0)),
            pl.BlockSpec((1, EMB_SIZE), lambda p: (0, 0)),
            pl.BlockSpec((1, Z_DIM), lambda p: (0, 0)),
        ],
        out_specs=pl.BlockSpec((NUM_CODES, Z_DIM), lambda p: (p, 0)),
        out_shape=jax.ShapeDtypeStruct((P * NUM_CODES, Z_DIM), jnp.float32),
        compiler_params=pltpu.CompilerParams(
            dimension_semantics=("arbitrary",)),
    )(codebook, fc_wpt, mu_wt, fc_b[None, :], mu_b[None, :])

    mall_cs = mall.reshape(P * NUM_CODES, NS, COLS).transpose(1, 2, 0).reshape(-1)

    sc_gather = pl.kernel(
        _sc_gather_body,
        out_type=jax.ShapeDtypeStruct((NS, COLS, BS), jnp.float32),
        mesh=plsc.VectorSubcoreMesh(core_axis_name="c", subcore_axis_name="s"),
        scratch_types=[
            pltpu.VMEM((P * NUM_CODES * COLS,), jnp.float32),
            pltpu.VMEM((P, CCH), jnp.int32),
            pltpu.VMEM((COLS, CCH), jnp.float32),
        ],
        compiler_params=pltpu.CompilerParams(needs_layout_passes=False),
    )
    out3 = sc_gather(eidx.T, mall_cs)
    mu = out3.transpose(2, 0, 1).reshape(BS, Z_DIM)

    return mu, cmt.reshape(()), perp.reshape(())


# speedup vs baseline: 3.6021x; 1.0105x over previous
import functools

import jax
import jax.numpy as jnp
from jax import lax
from jax.experimental import pallas as pl
from jax.experimental.pallas import tpu as pltpu
from jax.experimental.pallas import tpu_sc as plsc

BS = 16384
L = 32
P = 8
KSZ = 4
NUM_CH = 64
EMB_SIZE = 512
Z_DIM = 256
NUM_CODES = 128
COMMIT = 0.25

BLK = 2048
N_BLK = BS // BLK

NC = 2
NS = 16
LANES = 16
NW = NC * NS
B_PER_W = BS // NW
CH = 16
N_CHUNK = B_PER_W // CH
IDX_PER_CH = CH * P


def _main_body(fn_ref, valid_ref, wc_ref, cb_tiled_ref, codebook_ref, cbt_ref,
               bones_ref, ones_row_ref,
               eidx_ref, cmt_ref, perp_ref,
               hist_ref, acc_ref):
    i = pl.program_id(0)

    @pl.when(i == 0)
    def _init():
        hist_ref[...] = jnp.zeros_like(hist_ref)
        acc_ref[0] = 0.0
        acc_ref[1] = 0.0

    fn = fn_ref[...]
    valid = valid_ref[...]
    zbig = jnp.maximum(
        jnp.dot(fn, wc_ref[...], preferred_element_type=jnp.float32)
        + cb_tiled_ref[...], 0.0)

    codebook = codebook_ref[...]
    cbn2 = jnp.sum(codebook * codebook, axis=1)[None, :]
    cbtm2 = cbt_ref[...] * -2.0
    iotaf = jax.lax.broadcasted_iota(
        jnp.int32, (BLK, NUM_CODES), 1).astype(jnp.float32)

    zn2_all = jnp.dot(zbig * zbig, bones_ref[...],
                      preferred_element_type=jnp.float32)
    zn2_row = jnp.sum(zn2_all, axis=1, keepdims=True)

    oh_sum = jnp.zeros((BLK, NUM_CODES), jnp.float32)
    dvec = jnp.zeros((BLK, 1), jnp.float32)
    cols = []
    for p in range(P):
        z_p = zbig[:, p * NUM_CH:(p + 1) * NUM_CH]
        dist = jnp.dot(z_p, cbtm2, preferred_element_type=jnp.float32) + cbn2
        dmin = jnp.min(dist, axis=1, keepdims=True)
        eqm = dist == dmin
        idxf = jnp.min(jnp.where(eqm, iotaf, 1e9), axis=1, keepdims=True)
        oh_sum = oh_sum + eqm.astype(jnp.float32)
        dvec = dvec + dmin
        cols.append(idxf.astype(jnp.int32) + p * NUM_CODES)
    eidx_ref[...] = jnp.concatenate(cols, axis=1)

    hist_ref[...] += jnp.dot(ones_row_ref[...], oh_sum,
                             preferred_element_type=jnp.float32)
    acc_ref[0] += jnp.sum((dvec + zn2_row) * valid)
    acc_ref[1] += jnp.sum(valid)

    @pl.when(i == N_BLK - 1)
    def _fini():
        denom = jnp.maximum(acc_ref[1] * (P * NUM_CH), 1.0)
        cmt_ref[...] = jnp.full((1, 1), COMMIT * acc_ref[0] / denom,
                                jnp.float32)
        avgp = hist_ref[...] / float(BS * P)
        perp_ref[...] = jnp.full(
            (1, 1), jnp.exp(-jnp.sum(avgp * jnp.log(avgp + 1e-10))),
            jnp.float32)


def _mall_body(cb_ref, fc_wpt_ref, mu_wt_ref, fc_b_ref, mu_b_ref, mall_ref):
    t = jnp.dot(cb_ref[...], fc_wpt_ref[0], preferred_element_type=jnp.float32)
    m = jnp.dot(t, mu_wt_ref[...], preferred_element_type=jnp.float32)
    bias = (jnp.dot(fc_b_ref[...], mu_wt_ref[...],
                    preferred_element_type=jnp.float32)
            + mu_b_ref[...]) * (1.0 / P)
    mall_ref[...] = m + bias


HALF = BS // NC
CCH = 2048
N_CCH = HALF // CCH
COLS = Z_DIM // NS


def _sc_gather_body(eidx_hbm, mall_hbm, out_hbm, table_v, eidx_v, out_v):
    core = lax.axis_index("c")
    tile = lax.axis_index("s")
    sample_base = core * HALF

    pltpu.sync_copy(
        mall_hbm.at[pl.ds(tile * (P * NUM_CODES * COLS), P * NUM_CODES * COLS)],
        table_v)

    def chunk(k, carry):
        s0 = sample_base + k * CCH
        pltpu.sync_copy(eidx_hbm.at[:, pl.ds(s0, CCH)], eidx_v)

        @plsc.parallel_loop(0, CCH // LANES, 1)
        def _group(g):
            ev = [eidx_v[p, pl.ds(g * LANES, LANES)] for p in range(P)]
            for c in range(COLS):
                cbase = c * (P * NUM_CODES)
                acc = plsc.load_gather(table_v, [ev[0] + cbase])
                for p in range(1, P):
                    acc = acc + plsc.load_gather(table_v, [ev[p] + cbase])
                out_v[c, pl.ds(g * LANES, LANES)] = acc

        pltpu.sync_copy(out_v, out_hbm.at[tile, :, pl.ds(s0, CCH)])
        return carry

    lax.fori_loop(0, N_CCH, chunk, 0)


@jax.jit
def kernel(fn, track_pad_mask, conv_w, conv_b, fc_w, fc_b, mu_w, mu_b, codebook):
    valid = 1.0 - track_pad_mask.astype(jnp.float32)
    w_kc = conv_w[:, 0, :].T
    wc = jnp.kron(jnp.eye(P, dtype=jnp.float32), w_kc)
    cb_tiled = jnp.tile(conv_b, P)[None, :]
    fc_wpt = fc_w.reshape(EMB_SIZE, NUM_CH, P).transpose(2, 1, 0)
    mu_wt = mu_w.T

    eidx, cmt, perp = pl.pallas_call(
        _main_body,
        grid=(N_BLK,),
        in_specs=[
            pl.BlockSpec((BLK, L), lambda i: (i, 0)),
            pl.BlockSpec((BLK, 1), lambda i: (i, 0)),
            pl.BlockSpec((L, EMB_SIZE), lambda i: (0, 0)),
            pl.BlockSpec((1, EMB_SIZE), lambda i: (0, 0)),
            pl.BlockSpec((NUM_CODES, NUM_CH), lambda i: (0, 0)),
            pl.BlockSpec((NUM_CH, NUM_CODES), lambda i: (0, 0)),
            pl.BlockSpec((EMB_SIZE, P), lambda i: (0, 0)),
            pl.BlockSpec((1, BLK), lambda i: (0, 0)),
        ],
        out_specs=[
            pl.BlockSpec((BLK, P), lambda i: (i, 0)),
            pl.BlockSpec((1, 1), lambda i: (0, 0)),
            pl.BlockSpec((1, 1), lambda i: (0, 0)),
        ],
        out_shape=[
            jax.ShapeDtypeStruct((BS, P), jnp.int32),
            jax.ShapeDtypeStruct((1, 1), jnp.float32),
            jax.ShapeDtypeStruct((1, 1), jnp.float32),
        ],
        scratch_shapes=[
            pltpu.VMEM((1, NUM_CODES), jnp.float32),
            pltpu.SMEM((2,), jnp.float32),
        ],
        compiler_params=pltpu.CompilerParams(
            dimension_semantics=("arbitrary",)),
    )(fn, valid, wc, cb_tiled, codebook, codebook.T,
      jnp.kron(jnp.eye(P, dtype=jnp.float32), jnp.ones((NUM_CH, 1), jnp.float32)),
      jnp.ones((1, BLK), jnp.float32))

    mall = pl.pallas_call(
        _mall_body,
        grid=(P,),
        in_specs=[
            pl.BlockSpec((NUM_CODES, NUM_CH), lambda p: (0, 0)),
            pl.BlockSpec((1, NUM_CH, EMB_SIZE), lambda p: (p, 0, 0)),
            pl.BlockSpec((EMB_SIZE, Z_DIM), lambda p: (0, 0)),
            pl.BlockSpec((1, EMB_SIZE), lambda p: (0, 0)),
            pl.BlockSpec((1, Z_DIM), lambda p: (0, 0)),
        ],
        out_specs=pl.BlockSpec((NUM_CODES, Z_DIM), lambda p: (p, 0)),
        out_shape=jax.ShapeDtypeStruct((P * NUM_CODES, Z_DIM), jnp.float32),
        compiler_params=pltpu.CompilerParams(
            dimension_semantics=("arbitrary",)),
    )(codebook, fc_wpt, mu_wt, fc_b[None, :], mu_b[None, :])

    mall_cs = mall.reshape(P * NUM_CODES, NS, COLS).transpose(1, 2, 0).reshape(-1)

    sc_gather = pl.kernel(
        _sc_gather_body,
        out_type=jax.ShapeDtypeStruct((NS, COLS, BS), jnp.float32),
        mesh=plsc.VectorSubcoreMesh(core_axis_name="c", subcore_axis_name="s"),
        scratch_types=[
            pltpu.VMEM((P * NUM_CODES * COLS,), jnp.float32),
            pltpu.VMEM((P, CCH), jnp.int32),
            pltpu.VMEM((COLS, CCH), jnp.float32),
        ],
        compiler_params=pltpu.CompilerParams(needs_layout_passes=False),
    )
    out3 = sc_gather(eidx.T, mall_cs)
    mu = out3.transpose(2, 0, 1).reshape(BS, Z_DIM)

    return mu, cmt.reshape(()), perp.reshape(())


# speedup vs baseline: 3.6501x; 1.0133x over previous
import functools

import jax
import jax.numpy as jnp
from jax import lax
from jax.experimental import pallas as pl
from jax.experimental.pallas import tpu as pltpu
from jax.experimental.pallas import tpu_sc as plsc

BS = 16384
L = 32
P = 8
KSZ = 4
NUM_CH = 64
EMB_SIZE = 512
Z_DIM = 256
NUM_CODES = 128
COMMIT = 0.25

BLK = 2048
N_BLK = BS // BLK

NC = 2
NS = 16
LANES = 16
NW = NC * NS
B_PER_W = BS // NW
CH = 16
N_CHUNK = B_PER_W // CH
IDX_PER_CH = CH * P


HALF_B = BS // 2
N_BLK_H = HALF_B // BLK


def _main_body(fn_ref, valid_ref, wc_ref, cb_tiled_ref, codebook_ref, cbt_ref,
               bones_ref, ones_row_ref,
               eidx_ref, hist_out_ref, ds_ref, vs_ref,
               hist_ref, acc_ref):
    i = pl.program_id(0)

    @pl.when(i == 0)
    def _init():
        hist_ref[...] = jnp.zeros_like(hist_ref)
        acc_ref[0] = 0.0
        acc_ref[1] = 0.0

    fn = fn_ref[...]
    valid = valid_ref[...]
    zbig = jnp.maximum(
        jnp.dot(fn, wc_ref[...], preferred_element_type=jnp.float32)
        + cb_tiled_ref[...], 0.0)

    codebook = codebook_ref[...]
    cbn2 = jnp.sum(codebook * codebook, axis=1)[None, :]
    cbtm2 = cbt_ref[...] * -2.0
    iotaf = jax.lax.broadcasted_iota(
        jnp.int32, (BLK, NUM_CODES), 1).astype(jnp.float32)

    zn2_all = jnp.dot(zbig * zbig, bones_ref[...],
                      preferred_element_type=jnp.float32)
    zn2_row = jnp.sum(zn2_all, axis=1, keepdims=True)

    oh_sum = jnp.zeros((BLK, NUM_CODES), jnp.float32)
    dvec = jnp.zeros((BLK, 1), jnp.float32)
    cols = []
    for p in range(P):
        z_p = zbig[:, p * NUM_CH:(p + 1) * NUM_CH]
        dist = jnp.dot(z_p, cbtm2, preferred_element_type=jnp.float32) + cbn2
        dmin = jnp.min(dist, axis=1, keepdims=True)
        eqm = dist == dmin
        idxf = jnp.min(jnp.where(eqm, iotaf, 1e9), axis=1, keepdims=True)
        oh_sum = oh_sum + eqm.astype(jnp.float32)
        dvec = dvec + dmin
        cols.append(idxf.astype(jnp.int32) + p * NUM_CODES)
    eidx_ref[...] = jnp.concatenate(cols, axis=1)

    hist_ref[...] += jnp.dot(ones_row_ref[...], oh_sum,
                             preferred_element_type=jnp.float32)
    acc_ref[0] += jnp.sum((dvec + zn2_row) * valid)
    acc_ref[1] += jnp.sum(valid)

    @pl.when(i == N_BLK_H - 1)
    def _fini():
        hist_out_ref[...] = hist_ref[...]
        ds_ref[...] = jnp.full((1, 1), acc_ref[0], jnp.float32)
        vs_ref[...] = jnp.full((1, 1), acc_ref[1], jnp.float32)


def _scalars_body(h0_ref, h1_ref, d0_ref, d1_ref, v0_ref, v1_ref,
                  cmt_ref, perp_ref):
    dsum = d0_ref[0, 0] + d1_ref[0, 0]
    vsum = v0_ref[0, 0] + v1_ref[0, 0]
    denom = jnp.maximum(vsum * (P * NUM_CH), 1.0)
    cmt_ref[...] = jnp.full((1, 1), COMMIT * dsum / denom, jnp.float32)
    avgp = (h0_ref[...] + h1_ref[...]) / float(BS * P)
    perp_ref[...] = jnp.full(
        (1, 1), jnp.exp(-jnp.sum(avgp * jnp.log(avgp + 1e-10))), jnp.float32)


def _mall_body(cb_ref, fc_wpt_ref, mu_wt_ref, fc_b_ref, mu_b_ref, mall_ref):
    t = jnp.dot(cb_ref[...], fc_wpt_ref[0], preferred_element_type=jnp.float32)
    m = jnp.dot(t, mu_wt_ref[...], preferred_element_type=jnp.float32)
    bias = (jnp.dot(fc_b_ref[...], mu_wt_ref[...],
                    preferred_element_type=jnp.float32)
            + mu_b_ref[...]) * (1.0 / P)
    mall_ref[...] = m + bias


PER_CORE = HALF_B // NC
CCH = 2048
N_CCH = PER_CORE // CCH
COLS = Z_DIM // NS


def _sc_gather_body(eidx_hbm, mall_hbm, out_hbm, table_v, eidx_v, out_v):
    core = lax.axis_index("c")
    tile = lax.axis_index("s")
    sample_base = core * PER_CORE

    pltpu.sync_copy(
        mall_hbm.at[pl.ds(tile * (P * NUM_CODES * COLS), P * NUM_CODES * COLS)],
        table_v)

    def chunk(k, carry):
        s0 = sample_base + k * CCH
        pltpu.sync_copy(eidx_hbm.at[:, pl.ds(s0, CCH)], eidx_v)

        @plsc.parallel_loop(0, CCH // LANES, 1)
        def _group(g):
            ev = [eidx_v[p, pl.ds(g * LANES, LANES)] for p in range(P)]
            for c in range(COLS):
                cbase = c * (P * NUM_CODES)
                acc = plsc.load_gather(table_v, [ev[0] + cbase])
                for p in range(1, P):
                    acc = acc + plsc.load_gather(table_v, [ev[p] + cbase])
                out_v[c, pl.ds(g * LANES, LANES)] = acc

        pltpu.sync_copy(out_v, out_hbm.at[tile, :, pl.ds(s0, CCH)])
        return carry

    lax.fori_loop(0, N_CCH, chunk, 0)


@jax.jit
def kernel(fn, track_pad_mask, conv_w, conv_b, fc_w, fc_b, mu_w, mu_b, codebook):
    valid = 1.0 - track_pad_mask.astype(jnp.float32)
    w_kc = conv_w[:, 0, :].T
    wc = jnp.kron(jnp.eye(P, dtype=jnp.float32), w_kc)
    cb_tiled = jnp.tile(conv_b, P)[None, :]
    fc_wpt = fc_w.reshape(EMB_SIZE, NUM_CH, P).transpose(2, 1, 0)
    mu_wt = mu_w.T

    bones = jnp.kron(jnp.eye(P, dtype=jnp.float32),
                     jnp.ones((NUM_CH, 1), jnp.float32))
    ones_row = jnp.ones((1, BLK), jnp.float32)

    main_call = pl.pallas_call(
        _main_body,
        grid=(N_BLK_H,),
        in_specs=[
            pl.BlockSpec((BLK, L), lambda i: (i, 0)),
            pl.BlockSpec((BLK, 1), lambda i: (i, 0)),
            pl.BlockSpec((L, EMB_SIZE), lambda i: (0, 0)),
            pl.BlockSpec((1, EMB_SIZE), lambda i: (0, 0)),
            pl.BlockSpec((NUM_CODES, NUM_CH), lambda i: (0, 0)),
            pl.BlockSpec((NUM_CH, NUM_CODES), lambda i: (0, 0)),
            pl.BlockSpec((EMB_SIZE, P), lambda i: (0, 0)),
            pl.BlockSpec((1, BLK), lambda i: (0, 0)),
        ],
        out_specs=[
            pl.BlockSpec((BLK, P), lambda i: (i, 0)),
            pl.BlockSpec((1, NUM_CODES), lambda i: (0, 0)),
            pl.BlockSpec((1, 1), lambda i: (0, 0)),
            pl.BlockSpec((1, 1), lambda i: (0, 0)),
        ],
        out_shape=[
            jax.ShapeDtypeStruct((HALF_B, P), jnp.int32),
            jax.ShapeDtypeStruct((1, NUM_CODES), jnp.float32),
            jax.ShapeDtypeStruct((1, 1), jnp.float32),
            jax.ShapeDtypeStruct((1, 1), jnp.float32),
        ],
        scratch_shapes=[
            pltpu.VMEM((1, NUM_CODES), jnp.float32),
            pltpu.SMEM((2,), jnp.float32),
        ],
        compiler_params=pltpu.CompilerParams(
            dimension_semantics=("arbitrary",)),
    )

    eidx0, hist0, d0, v0 = main_call(
        fn[:HALF_B], valid[:HALF_B], wc, cb_tiled, codebook, codebook.T,
        bones, ones_row)
    eidx1, hist1, d1, v1 = main_call(
        fn[HALF_B:], valid[HALF_B:], wc, cb_tiled, codebook, codebook.T,
        bones, ones_row)

    cmt, perp = pl.pallas_call(
        _scalars_body,
        in_specs=[pl.BlockSpec((1, NUM_CODES), lambda: (0, 0)),
                  pl.BlockSpec((1, NUM_CODES), lambda: (0, 0)),
                  pl.BlockSpec((1, 1), lambda: (0, 0)),
                  pl.BlockSpec((1, 1), lambda: (0, 0)),
                  pl.BlockSpec((1, 1), lambda: (0, 0)),
                  pl.BlockSpec((1, 1), lambda: (0, 0))],
        out_specs=[pl.BlockSpec((1, 1), lambda: (0, 0)),
                   pl.BlockSpec((1, 1), lambda: (0, 0))],
        out_shape=[jax.ShapeDtypeStruct((1, 1), jnp.float32),
                   jax.ShapeDtypeStruct((1, 1), jnp.float32)],
    )(hist0, hist1, d0, d1, v0, v1)

    mall = pl.pallas_call(
        _mall_body,
        grid=(P,),
        in_specs=[
            pl.BlockSpec((NUM_CODES, NUM_CH), lambda p: (0, 0)),
            pl.BlockSpec((1, NUM_CH, EMB_SIZE), lambda p: (p, 0, 0)),
            pl.BlockSpec((EMB_SIZE, Z_DIM), lambda p: (0, 0)),
            pl.BlockSpec((1, EMB_SIZE), lambda p: (0, 0)),
            pl.BlockSpec((1, Z_DIM), lambda p: (0, 0)),
        ],
        out_specs=pl.BlockSpec((NUM_CODES, Z_DIM), lambda p: (p, 0)),
        out_shape=jax.ShapeDtypeStruct((P * NUM_CODES, Z_DIM), jnp.float32),
        compiler_params=pltpu.CompilerParams(
            dimension_semantics=("arbitrary",)),
    )(codebook, fc_wpt, mu_wt, fc_b[None, :], mu_b[None, :])

    mall_cs = mall.reshape(P * NUM_CODES, NS, COLS).transpose(1, 2, 0).reshape(-1)

    sc_gather = pl.kernel(
        _sc_gather_body,
        out_type=jax.ShapeDtypeStruct((NS, COLS, HALF_B), jnp.float32),
        mesh=plsc.VectorSubcoreMesh(core_axis_name="c", subcore_axis_name="s"),
        scratch_types=[
            pltpu.VMEM((P * NUM_CODES * COLS,), jnp.float32),
            pltpu.VMEM((P, CCH), jnp.int32),
            pltpu.VMEM((COLS, CCH), jnp.float32),
        ],
        compiler_params=pltpu.CompilerParams(needs_layout_passes=False),
    )
    out3_0 = sc_gather(eidx0.T, mall_cs)
    out3_1 = sc_gather(eidx1.T, mall_cs)
    mu = jnp.concatenate(
        [out3_0.transpose(2, 0, 1).reshape(HALF_B, Z_DIM),
         out3_1.transpose(2, 0, 1).reshape(HALF_B, Z_DIM)], axis=0)

    return mu, cmt.reshape(()), perp.reshape(())


# speedup vs baseline: 3.6888x; 1.0106x over previous
import functools

import jax
import jax.numpy as jnp
from jax import lax
from jax.experimental import pallas as pl
from jax.experimental.pallas import tpu as pltpu
from jax.experimental.pallas import tpu_sc as plsc

BS = 16384
L = 32
P = 8
KSZ = 4
NUM_CH = 64
EMB_SIZE = 512
Z_DIM = 256
NUM_CODES = 128
COMMIT = 0.25

BLK = 2048
N_BLK = BS // BLK

NC = 2
NS = 16
LANES = 16
NW = NC * NS
B_PER_W = BS // NW
CH = 16
N_CHUNK = B_PER_W // CH
IDX_PER_CH = CH * P


HALF_B = BS // 2
N_BLK_H = HALF_B // BLK


def _main_body(fn_ref, valid_ref, wc_ref, cb_tiled_ref, codebook_ref, cbt_ref,
               bones_ref, ones_row_ref,
               eidx_ref, hist_out_ref, ds_ref, vs_ref,
               hist_ref, acc_ref):
    i = pl.program_id(0)

    @pl.when(i == 0)
    def _init():
        hist_ref[...] = jnp.zeros_like(hist_ref)
        acc_ref[0] = 0.0
        acc_ref[1] = 0.0

    fn = fn_ref[...]
    valid = valid_ref[...]
    zbig = jnp.maximum(
        jnp.dot(fn, wc_ref[...], preferred_element_type=jnp.float32)
        + cb_tiled_ref[...], 0.0)

    codebook = codebook_ref[...]
    cbn2 = jnp.sum(codebook * codebook, axis=1)[None, :]
    cbtm2 = cbt_ref[...] * -2.0
    iotaf = jax.lax.broadcasted_iota(
        jnp.int32, (BLK, NUM_CODES), 1).astype(jnp.float32)

    zn2_all = jnp.dot(zbig * zbig, bones_ref[...],
                      preferred_element_type=jnp.float32)
    zn2_row = jnp.sum(zn2_all, axis=1, keepdims=True)

    oh_sum = jnp.zeros((BLK, NUM_CODES), jnp.float32)
    dvec = jnp.zeros((BLK, 1), jnp.float32)
    cols = []
    for p in range(P):
        z_p = zbig[:, p * NUM_CH:(p + 1) * NUM_CH]
        dist = jnp.dot(z_p, cbtm2, preferred_element_type=jnp.float32) + cbn2
        dmin = jnp.min(dist, axis=1, keepdims=True)
        idx = jnp.argmin(dist, axis=1).reshape(BLK, 1)
        oh_sum = oh_sum + (iotaf == idx.astype(jnp.float32))
        dvec = dvec + dmin
        cols.append(idx + p * NUM_CODES)
    eidx_ref[...] = jnp.concatenate(cols, axis=1)

    hist_ref[...] += jnp.dot(ones_row_ref[...], oh_sum,
                             preferred_element_type=jnp.float32)
    acc_ref[0] += jnp.sum((dvec + zn2_row) * valid)
    acc_ref[1] += jnp.sum(valid)

    @pl.when(i == N_BLK_H - 1)
    def _fini():
        hist_out_ref[...] = hist_ref[...]
        ds_ref[...] = jnp.full((1, 1), acc_ref[0], jnp.float32)
        vs_ref[...] = jnp.full((1, 1), acc_ref[1], jnp.float32)


def _scalars_body(h0_ref, h1_ref, d0_ref, d1_ref, v0_ref, v1_ref,
                  cmt_ref, perp_ref):
    dsum = d0_ref[0, 0] + d1_ref[0, 0]
    vsum = v0_ref[0, 0] + v1_ref[0, 0]
    denom = jnp.maximum(vsum * (P * NUM_CH), 1.0)
    cmt_ref[...] = jnp.full((1, 1), COMMIT * dsum / denom, jnp.float32)
    avgp = (h0_ref[...] + h1_ref[...]) / float(BS * P)
    perp_ref[...] = jnp.full(
        (1, 1), jnp.exp(-jnp.sum(avgp * jnp.log(avgp + 1e-10))), jnp.float32)


def _mall_body(cb_ref, fc_wpt_ref, mu_wt_ref, fc_b_ref, mu_b_ref, mall_ref):
    t = jnp.dot(cb_ref[...], fc_wpt_ref[0], preferred_element_type=jnp.float32)
    m = jnp.dot(t, mu_wt_ref[...], preferred_element_type=jnp.float32)
    bias = (jnp.dot(fc_b_ref[...], mu_wt_ref[...],
                    preferred_element_type=jnp.float32)
            + mu_b_ref[...]) * (1.0 / P)
    mall_ref[...] = m + bias


PER_CORE = HALF_B // NC
CCH = 2048
N_CCH = PER_CORE // CCH
COLS = Z_DIM // NS


def _sc_gather_body(eidx_hbm, mall_hbm, out_hbm, table_v, eidx_v, out_v):
    core = lax.axis_index("c")
    tile = lax.axis_index("s")
    sample_base = core * PER_CORE

    pltpu.sync_copy(
        mall_hbm.at[pl.ds(tile * (P * NUM_CODES * COLS), P * NUM_CODES * COLS)],
        table_v)

    def chunk(k, carry):
        s0 = sample_base + k * CCH
        pltpu.sync_copy(eidx_hbm.at[:, pl.ds(s0, CCH)], eidx_v)

        @plsc.parallel_loop(0, CCH // LANES, 1)
        def _group(g):
            ev = [eidx_v[p, pl.ds(g * LANES, LANES)] for p in range(P)]
            for c in range(COLS):
                cbase = c * (P * NUM_CODES)
                acc = plsc.load_gather(table_v, [ev[0] + cbase])
                for p in range(1, P):
                    acc = acc + plsc.load_gather(table_v, [ev[p] + cbase])
                out_v[c, pl.ds(g * LANES, LANES)] = acc

        pltpu.sync_copy(out_v, out_hbm.at[tile, :, pl.ds(s0, CCH)])
        return carry

    lax.fori_loop(0, N_CCH, chunk, 0)


@jax.jit
def kernel(fn, track_pad_mask, conv_w, conv_b, fc_w, fc_b, mu_w, mu_b, codebook):
    valid = 1.0 - track_pad_mask.astype(jnp.float32)
    w_kc = conv_w[:, 0, :].T
    wc = jnp.kron(jnp.eye(P, dtype=jnp.float32), w_kc)
    cb_tiled = jnp.tile(conv_b, P)[None, :]
    fc_wpt = fc_w.reshape(EMB_SIZE, NUM_CH, P).transpose(2, 1, 0)
    mu_wt = mu_w.T

    bones = jnp.kron(jnp.eye(P, dtype=jnp.float32),
                     jnp.ones((NUM_CH, 1), jnp.float32))
    ones_row = jnp.ones((1, BLK), jnp.float32)

    main_call = pl.pallas_call(
        _main_body,
        grid=(N_BLK_H,),
        in_specs=[
            pl.BlockSpec((BLK, L), lambda i: (i, 0)),
            pl.BlockSpec((BLK, 1), lambda i: (i, 0)),
            pl.BlockSpec((L, EMB_SIZE), lambda i: (0, 0)),
            pl.BlockSpec((1, EMB_SIZE), lambda i: (0, 0)),
            pl.BlockSpec((NUM_CODES, NUM_CH), lambda i: (0, 0)),
            pl.BlockSpec((NUM_CH, NUM_CODES), lambda i: (0, 0)),
            pl.BlockSpec((EMB_SIZE, P), lambda i: (0, 0)),
            pl.BlockSpec((1, BLK), lambda i: (0, 0)),
        ],
        out_specs=[
            pl.BlockSpec((BLK, P), lambda i: (i, 0)),
            pl.BlockSpec((1, NUM_CODES), lambda i: (0, 0)),
            pl.BlockSpec((1, 1), lambda i: (0, 0)),
            pl.BlockSpec((1, 1), lambda i: (0, 0)),
        ],
        out_shape=[
            jax.ShapeDtypeStruct((HALF_B, P), jnp.int32),
            jax.ShapeDtypeStruct((1, NUM_CODES), jnp.float32),
            jax.ShapeDtypeStruct((1, 1), jnp.float32),
            jax.ShapeDtypeStruct((1, 1), jnp.float32),
        ],
        scratch_shapes=[
            pltpu.VMEM((1, NUM_CODES), jnp.float32),
            pltpu.SMEM((2,), jnp.float32),
        ],
        compiler_params=pltpu.CompilerParams(
            dimension_semantics=("arbitrary",)),
    )

    eidx0, hist0, d0, v0 = main_call(
        fn[:HALF_B], valid[:HALF_B], wc, cb_tiled, codebook, codebook.T,
        bones, ones_row)
    eidx1, hist1, d1, v1 = main_call(
        fn[HALF_B:], valid[HALF_B:], wc, cb_tiled, codebook, codebook.T,
        bones, ones_row)

    cmt, perp = pl.pallas_call(
        _scalars_body,
        in_specs=[pl.BlockSpec((1, NUM_CODES), lambda: (0, 0)),
                  pl.BlockSpec((1, NUM_CODES), lambda: (0, 0)),
                  pl.BlockSpec((1, 1), lambda: (0, 0)),
                  pl.BlockSpec((1, 1), lambda: (0, 0)),
                  pl.BlockSpec((1, 1), lambda: (0, 0)),
                  pl.BlockSpec((1, 1), lambda: (0, 0))],
        out_specs=[pl.BlockSpec((1, 1), lambda: (0, 0)),
                   pl.BlockSpec((1, 1), lambda: (0, 0))],
        out_shape=[jax.ShapeDtypeStruct((1, 1), jnp.float32),
                   jax.ShapeDtypeStruct((1, 1), jnp.float32)],
    )(hist0, hist1, d0, d1, v0, v1)

    mall = pl.pallas_call(
        _mall_body,
        grid=(P,),
        in_specs=[
            pl.BlockSpec((NUM_CODES, NUM_CH), lambda p: (0, 0)),
            pl.BlockSpec((1, NUM_CH, EMB_SIZE), lambda p: (p, 0, 0)),
            pl.BlockSpec((EMB_SIZE, Z_DIM), lambda p: (0, 0)),
            pl.BlockSpec((1, EMB_SIZE), lambda p: (0, 0)),
            pl.BlockSpec((1, Z_DIM), lambda p: (0, 0)),
        ],
        out_specs=pl.BlockSpec((NUM_CODES, Z_DIM), lambda p: (p, 0)),
        out_shape=jax.ShapeDtypeStruct((P * NUM_CODES, Z_DIM), jnp.float32),
        compiler_params=pltpu.CompilerParams(
            dimension_semantics=("arbitrary",)),
    )(codebook, fc_wpt, mu_wt, fc_b[None, :], mu_b[None, :])

    mall_cs = mall.reshape(P * NUM_CODES, NS, COLS).transpose(1, 2, 0).reshape(-1)

    sc_gather = pl.kernel(
        _sc_gather_body,
        out_type=jax.ShapeDtypeStruct((NS, COLS, HALF_B), jnp.float32),
        mesh=plsc.VectorSubcoreMesh(core_axis_name="c", subcore_axis_name="s"),
        scratch_types=[
            pltpu.VMEM((P * NUM_CODES * COLS,), jnp.float32),
            pltpu.VMEM((P, CCH), jnp.int32),
            pltpu.VMEM((COLS, CCH), jnp.float32),
        ],
        compiler_params=pltpu.CompilerParams(needs_layout_passes=False),
    )
    out3_0 = sc_gather(eidx0.T, mall_cs)
    out3_1 = sc_gather(eidx1.T, mall_cs)
    mu = jnp.concatenate(
        [out3_0.transpose(2, 0, 1).reshape(HALF_B, Z_DIM),
         out3_1.transpose(2, 0, 1).reshape(HALF_B, Z_DIM)], axis=0)

    return mu, cmt.reshape(()), perp.reshape(())
